# baseline (device time: 60978 ns/iter reference)
import jax
import jax.numpy as jnp
from jax import lax
from jax.experimental import pallas as pl
from jax.experimental.pallas import tpu as pltpu

N_DEV = 4
EPS = 1e-5
BLOCK_M = 1024
BLOCK_C = 1024
LANES = 128


def _pack_rows(s, nrows):
    r_idx = lax.broadcasted_iota(jnp.int32, (nrows, LANES), 0)
    b_idx = lax.broadcasted_iota(jnp.int32, (nrows, LANES), 1)
    masked = s * (r_idx % LANES == b_idx).astype(jnp.float32)
    a_idx = lax.broadcasted_iota(jnp.int32, (nrows // LANES, nrows), 0)
    rr_idx = lax.broadcasted_iota(jnp.int32, (nrows // LANES, nrows), 1)
    sel = (rr_idx // LANES == a_idx).astype(jnp.float32)
    return jax.lax.dot(sel, masked, preferred_element_type=jnp.float32)


def _unpack_rows(p, nrows):
    r_idx = lax.broadcasted_iota(jnp.int32, (nrows, LANES), 0)
    b_idx = lax.broadcasted_iota(jnp.int32, (nrows, LANES), 1)
    a_idx = lax.broadcasted_iota(jnp.int32, (nrows, nrows // LANES), 1)
    rr_idx = lax.broadcasted_iota(jnp.int32, (nrows, nrows // LANES), 0)
    sel = (rr_idx // LANES == a_idx).astype(jnp.float32)
    w = jax.lax.dot(sel, p, preferred_element_type=jnp.float32)
    w = w * (r_idx % LANES == b_idx).astype(jnp.float32)
    return jnp.sum(w, axis=1, keepdims=True)


def _round_rows(rnd, tot_rows):
    half = tot_rows // 2
    return (0, half) if rnd == 0 else (half, tot_rows - half)


def _start_round_sends(rnd, me, acc_ref, comm_ref, send_sems, recv_sems):
    start, size = _round_rows(rnd, acc_ref.shape[0])
    sends = []
    for d in range(1, N_DEV):
        peer = (me + d) % N_DEV
        rdma = pltpu.make_async_remote_copy(
            src_ref=acc_ref.at[pl.ds(start, size), :],
            dst_ref=comm_ref.at[me, pl.ds(start, size), :],
            send_sem=send_sems.at[rnd, d - 1],
            recv_sem=recv_sems.at[rnd, me],
            device_id=(peer,),
            device_id_type=pl.DeviceIdType.MESH,
        )
        rdma.start()
        sends.append(rdma)
    return sends


def _partial_exchange_body(x_ref, rout_ref, xb_ref,
                           acc_ref, comm_ref, send_sems, recv_sems):
    i = pl.program_id(0)
    nblk = pl.num_programs(0)
    me = lax.axis_index("i")
    pk = BLOCK_M // LANES

    @pl.when(i == 0)
    def _():
        barrier = pltpu.get_barrier_semaphore()
        for d in range(1, N_DEV):
            peer = (me + d) % N_DEV
            pl.semaphore_signal(
                barrier, inc=1,
                device_id=(peer,), device_id_type=pl.DeviceIdType.MESH,
            )
        pl.semaphore_wait(barrier, N_DEV - 1)

    xf = x_ref[...]
    s = jnp.sum(xf * xf, axis=1, keepdims=True)
    acc_ref[pl.ds(i * pk, pk), :] = _pack_rows(s, BLOCK_M)
    xb_ref[...] = xf.astype(jnp.bfloat16)

    @pl.when(i == nblk // 2 - 1)
    def _():
        _start_round_sends(0, me, acc_ref, comm_ref, send_sems, recv_sems)

    @pl.when(i == nblk - 1)
    def _():
        _start_round_sends(1, me, acc_ref, comm_ref, send_sems, recv_sems)

        for rnd in range(2):
            start, size = _round_rows(rnd, acc_ref.shape[0])
            for d in range(1, N_DEV):
                src = (me - d + N_DEV) % N_DEV
                recv = pltpu.make_async_remote_copy(
                    src_ref=acc_ref.at[pl.ds(start, size), :],
                    dst_ref=comm_ref.at[src, pl.ds(start, size), :],
                    send_sem=send_sems.at[rnd, 0],
                    recv_sem=recv_sems.at[rnd, src],
                    device_id=(me,),
                    device_id_type=pl.DeviceIdType.MESH,
                )
                recv.wait_recv()

        total = acc_ref[...]
        for peer in range(N_DEV):
            total = total + jnp.where(me == peer, 0.0, comm_ref[peer, :, :])

        for rnd in range(2):
            start, size = _round_rows(rnd, acc_ref.shape[0])
            for d in range(1, N_DEV):
                snd = pltpu.make_async_remote_copy(
                    src_ref=acc_ref.at[pl.ds(start, size), :],
                    dst_ref=comm_ref.at[me, pl.ds(start, size), :],
                    send_sem=send_sems.at[rnd, d - 1],
                    recv_sem=recv_sems.at[rnd, me],
                    device_id=((me + d) % N_DEV,),
                    device_id_type=pl.DeviceIdType.MESH,
                )
                snd.wait_send()

        rout_ref[...] = lax.rsqrt(total * (1.0 / (N_DEV * 2048.0)) + EPS)


def _scale_body(xb_ref, r_ref, g_ref, out_ref):
    u = _unpack_rows(r_ref[...], BLOCK_C).astype(jnp.bfloat16)
    out_ref[...] = xb_ref[...] * u * g_ref[...].astype(jnp.bfloat16)


def kernel(x, gamma):
    m, n_loc = x.shape
    nblk = m // BLOCK_M
    pk = BLOCK_M // LANES

    rrms, xb = pl.pallas_call(
        _partial_exchange_body,
        grid=(nblk,),
        out_shape=(
            jax.ShapeDtypeStruct((m // LANES, LANES), jnp.float32),
            jax.ShapeDtypeStruct((m, n_loc), jnp.bfloat16),
        ),
        in_specs=[
            pl.BlockSpec((BLOCK_M, n_loc), lambda i: (i, 0),
                         memory_space=pltpu.VMEM),
        ],
        out_specs=(
            pl.BlockSpec((m // LANES, LANES), lambda i: (0, 0),
                         memory_space=pltpu.VMEM),
            pl.BlockSpec((BLOCK_M, n_loc), lambda i: (i, 0),
                         memory_space=pltpu.VMEM),
        ),
        scratch_shapes=[
            pltpu.VMEM((m // LANES, LANES), jnp.float32),
            pltpu.VMEM((N_DEV, m // LANES, LANES), jnp.float32),
            pltpu.SemaphoreType.DMA((2, N_DEV - 1)),
            pltpu.SemaphoreType.DMA((2, N_DEV)),
        ],
        compiler_params=pltpu.CompilerParams(
            collective_id=0, vmem_limit_bytes=56 * 1024 * 1024
        ),
    )(x)

    g2 = gamma.reshape(1, n_loc)

    out = pl.pallas_call(
        _scale_body,
        grid=(m // BLOCK_C,),
        out_shape=jax.ShapeDtypeStruct((m, n_loc), jnp.bfloat16),
        in_specs=[
            pl.BlockSpec((BLOCK_C, n_loc), lambda i: (i, 0),
                         memory_space=pltpu.VMEM),
            pl.BlockSpec((BLOCK_C // LANES, LANES), lambda i: (i, 0),
                         memory_space=pltpu.VMEM),
            pl.BlockSpec((1, n_loc), lambda i: (0, 0),
                         memory_space=pltpu.VMEM),
        ],
        out_specs=pl.BlockSpec((BLOCK_C, n_loc), lambda i: (i, 0),
                               memory_space=pltpu.VMEM),
        compiler_params=pltpu.CompilerParams(
            vmem_limit_bytes=56 * 1024 * 1024
        ),
    )(xb, rrms, g2)
    return out


# device time: 60286 ns/iter; 1.0115x vs baseline; 1.0115x over previous
import jax
import jax.numpy as jnp
from jax import lax
from jax.experimental import pallas as pl
from jax.experimental.pallas import tpu as pltpu

N_DEV = 4
EPS = 1e-5
BLOCK_M = 1024
BLOCK_C = 1024
LANES = 128


def _pack_rows(s, nrows):
    r_idx = lax.broadcasted_iota(jnp.int32, (nrows, LANES), 0)
    b_idx = lax.broadcasted_iota(jnp.int32, (nrows, LANES), 1)
    masked = s * (r_idx % LANES == b_idx).astype(jnp.float32)
    a_idx = lax.broadcasted_iota(jnp.int32, (nrows // LANES, nrows), 0)
    rr_idx = lax.broadcasted_iota(jnp.int32, (nrows // LANES, nrows), 1)
    sel = (rr_idx // LANES == a_idx).astype(jnp.float32)
    return jax.lax.dot(sel, masked, preferred_element_type=jnp.float32)


def _unpack_rows(p, nrows):
    r_idx = lax.broadcasted_iota(jnp.int32, (nrows, LANES), 0)
    b_idx = lax.broadcasted_iota(jnp.int32, (nrows, LANES), 1)
    a_idx = lax.broadcasted_iota(jnp.int32, (nrows, nrows // LANES), 1)
    rr_idx = lax.broadcasted_iota(jnp.int32, (nrows, nrows // LANES), 0)
    sel = (rr_idx // LANES == a_idx).astype(jnp.float32)
    w = jax.lax.dot(sel, p, preferred_element_type=jnp.float32)
    w = w * (r_idx % LANES == b_idx).astype(jnp.float32)
    return jnp.sum(w, axis=1, keepdims=True)


def _round_rows(rnd, tot_rows):
    half = tot_rows // 2
    return (0, half) if rnd == 0 else (half, tot_rows - half)


def _start_round_sends(rnd, me, acc_ref, comm_ref, send_sems, recv_sems):
    start, size = _round_rows(rnd, acc_ref.shape[0])
    sends = []
    for d in range(1, N_DEV):
        peer = (me + d) % N_DEV
        rdma = pltpu.make_async_remote_copy(
            src_ref=acc_ref.at[pl.ds(start, size), :],
            dst_ref=comm_ref.at[me, pl.ds(start, size), :],
            send_sem=send_sems.at[rnd, d - 1],
            recv_sem=recv_sems.at[rnd, me],
            device_id=(peer,),
            device_id_type=pl.DeviceIdType.MESH,
        )
        rdma.start()
        sends.append(rdma)
    return sends


def _partial_exchange_body(x_ref, rout_ref, xb_ref,
                           acc_ref, comm_ref, send_sems, recv_sems):
    i = pl.program_id(0)
    nblk = pl.num_programs(0)
    me = lax.axis_index("i")
    pk = BLOCK_M // LANES

    @pl.when(i == 0)
    def _():
        barrier = pltpu.get_barrier_semaphore()
        for d in range(1, N_DEV):
            peer = (me + d) % N_DEV
            pl.semaphore_signal(
                barrier, inc=1,
                device_id=(peer,), device_id_type=pl.DeviceIdType.MESH,
            )

    xf = x_ref[...]
    s = jnp.sum(xf * xf, axis=1, keepdims=True)
    acc_ref[pl.ds(i * pk, pk), :] = _pack_rows(s, BLOCK_M)
    xb_ref[...] = xf.astype(jnp.bfloat16)

    @pl.when(i == nblk // 2 - 1)
    def _():
        pl.semaphore_wait(pltpu.get_barrier_semaphore(), N_DEV - 1)
        _start_round_sends(0, me, acc_ref, comm_ref, send_sems, recv_sems)

    @pl.when(i == nblk - 1)
    def _():
        _start_round_sends(1, me, acc_ref, comm_ref, send_sems, recv_sems)

        for rnd in range(2):
            start, size = _round_rows(rnd, acc_ref.shape[0])
            for d in range(1, N_DEV):
                src = (me - d + N_DEV) % N_DEV
                recv = pltpu.make_async_remote_copy(
                    src_ref=acc_ref.at[pl.ds(start, size), :],
                    dst_ref=comm_ref.at[src, pl.ds(start, size), :],
                    send_sem=send_sems.at[rnd, 0],
                    recv_sem=recv_sems.at[rnd, src],
                    device_id=(me,),
                    device_id_type=pl.DeviceIdType.MESH,
                )
                recv.wait_recv()

        total = acc_ref[...]
        for peer in range(N_DEV):
            total = total + jnp.where(me == peer, 0.0, comm_ref[peer, :, :])

        for rnd in range(2):
            start, size = _round_rows(rnd, acc_ref.shape[0])
            for d in range(1, N_DEV):
                snd = pltpu.make_async_remote_copy(
                    src_ref=acc_ref.at[pl.ds(start, size), :],
                    dst_ref=comm_ref.at[me, pl.ds(start, size), :],
                    send_sem=send_sems.at[rnd, d - 1],
                    recv_sem=recv_sems.at[rnd, me],
                    device_id=((me + d) % N_DEV,),
                    device_id_type=pl.DeviceIdType.MESH,
                )
                snd.wait_send()

        rout_ref[...] = lax.rsqrt(total * (1.0 / (N_DEV * 2048.0)) + EPS)


def _scale_body(xb_ref, r_ref, g_ref, out_ref):
    u = _unpack_rows(r_ref[...], BLOCK_C).astype(jnp.bfloat16)
    out_ref[...] = xb_ref[...] * u * g_ref[...].astype(jnp.bfloat16)


def kernel(x, gamma):
    m, n_loc = x.shape
    nblk = m // BLOCK_M
    pk = BLOCK_M // LANES

    rrms, xb = pl.pallas_call(
        _partial_exchange_body,
        grid=(nblk,),
        out_shape=(
            jax.ShapeDtypeStruct((m // LANES, LANES), jnp.float32),
            jax.ShapeDtypeStruct((m, n_loc), jnp.bfloat16),
        ),
        in_specs=[
            pl.BlockSpec((BLOCK_M, n_loc), lambda i: (i, 0),
                         memory_space=pltpu.VMEM),
        ],
        out_specs=(
            pl.BlockSpec((m // LANES, LANES), lambda i: (0, 0),
                         memory_space=pltpu.VMEM),
            pl.BlockSpec((BLOCK_M, n_loc), lambda i: (i, 0),
                         memory_space=pltpu.VMEM),
        ),
        scratch_shapes=[
            pltpu.VMEM((m // LANES, LANES), jnp.float32),
            pltpu.VMEM((N_DEV, m // LANES, LANES), jnp.float32),
            pltpu.SemaphoreType.DMA((2, N_DEV - 1)),
            pltpu.SemaphoreType.DMA((2, N_DEV)),
        ],
        compiler_params=pltpu.CompilerParams(
            collective_id=0, vmem_limit_bytes=56 * 1024 * 1024
        ),
    )(x)

    g2 = gamma.reshape(1, n_loc)

    out = pl.pallas_call(
        _scale_body,
        grid=(m // BLOCK_C,),
        out_shape=jax.ShapeDtypeStruct((m, n_loc), jnp.bfloat16),
        in_specs=[
            pl.BlockSpec((BLOCK_C, n_loc), lambda i: (i, 0),
                         memory_space=pltpu.VMEM),
            pl.BlockSpec((BLOCK_C // LANES, LANES), lambda i: (i, 0),
                         memory_space=pltpu.VMEM),
            pl.BlockSpec((1, n_loc), lambda i: (0, 0),
                         memory_space=pltpu.VMEM),
        ],
        out_specs=pl.BlockSpec((BLOCK_C, n_loc), lambda i: (i, 0),
                               memory_space=pltpu.VMEM),
        compiler_params=pltpu.CompilerParams(
            vmem_limit_bytes=56 * 1024 * 1024
        ),
    )(xb, rrms, g2)
    return out
